# Initial kernel scaffold; baseline (speedup 1.0000x reference)
import jax
import jax.numpy as jnp
from jax.experimental import pallas as pl


def kernel(X, lens, emb_table, W, b):
    raise NotImplementedError("write your pallas kernel here")


# speedup vs baseline: 9.5277x; 9.5277x over previous
import functools

import jax
import jax.numpy as jnp
from jax import lax
from jax.experimental import pallas as pl
from jax.experimental.pallas import tpu as pltpu
from jax.experimental.pallas import tpu_sc as plsc

_B = 4096
_L = 50
_V = 100000
_D = 64
_NC = 2
_NS = 16
_NW = _NC * _NS
_RPW = _B // _NW
_NG = _RPW // 16
_VB = 2000
_NVB = _V // _VB


def _tc_matvec_body(x_ref, w_ref, o_ref):
    x = x_ref[...]
    w = w_ref[...]
    o = lax.dot_general(w, x, (((1,), (1,)), ((), ())),
                        preferred_element_type=jnp.float32)
    o_ref[...] = o.reshape(1, 1, _VB)


def _tc_matvec(emb_table, W):
    out = pl.pallas_call(
        _tc_matvec_body,
        grid=(_NVB,),
        in_specs=[
            pl.BlockSpec((_VB, _D), lambda i: (i, 0)),
            pl.BlockSpec((1, _D), lambda i: (0, 0)),
        ],
        out_specs=pl.BlockSpec((1, 1, _VB), lambda i: (i, 0, 0)),
        out_shape=jax.ShapeDtypeStruct((_NVB, 1, _VB), jnp.float32),
    )(emb_table, W)
    return out.reshape(_V)


def _sc_pool_body(t_hbm, x_hbm, lens_hbm, b_hbm, out_hbm,
                  t_v, x_v, lens_v, b_v, out_v):
    c = lax.axis_index("c")
    s = lax.axis_index("s")
    wid = s * _NC + c
    base = wid * _RPW
    pltpu.sync_copy(t_hbm, t_v)
    pltpu.sync_copy(x_hbm.at[wid], x_v)
    pltpu.sync_copy(lens_hbm.at[pl.ds(base, _RPW)], lens_v)
    pltpu.sync_copy(b_hbm, b_v)
    bvec = b_v[...]
    lens_g = [lens_v[pl.ds(g * 16, 16)] for g in range(_NG)]

    def body(j, accs):
        new = []
        for g in range(_NG):
            idx = x_v[j, pl.ds(g * 16, 16)]
            vals = plsc.load_gather(t_v, [idx])
            mask = j < lens_g[g]
            new.append(accs[g] + jnp.where(mask, vals, 0.0))
        return tuple(new)

    accs = lax.fori_loop(
        0, _L, body,
        tuple(jnp.zeros((16,), jnp.float32) for _ in range(_NG)))
    for g in range(_NG):
        score = accs[g] / lens_g[g].astype(jnp.float32) + bvec
        out_v[pl.ds(g * 16, 16)] = 1.0 / (1.0 + jnp.exp(-score))
    pltpu.sync_copy(out_v, out_hbm.at[pl.ds(base, _RPW)])


_sc_pool = pl.kernel(
    _sc_pool_body,
    out_type=jax.ShapeDtypeStruct((_B,), jnp.float32),
    mesh=plsc.VectorSubcoreMesh(core_axis_name="c", subcore_axis_name="s",
                                num_cores=_NC, num_subcores=_NS),
    compiler_params=pltpu.CompilerParams(needs_layout_passes=False),
    scratch_types=[
        pltpu.VMEM((_V,), jnp.float32),
        pltpu.VMEM((_L, _RPW), jnp.int32),
        pltpu.VMEM((_RPW,), jnp.int32),
        pltpu.VMEM((16,), jnp.float32),
        pltpu.VMEM((_RPW,), jnp.float32),
    ],
)


def kernel(X, lens, emb_table, W, b):
    t = _tc_matvec(emb_table, W)
    x3 = (X.astype(jnp.int32)
          .reshape(_NW, _RPW, _L)
          .transpose(0, 2, 1))
    lens_i = lens.astype(jnp.int32)
    b16 = jnp.broadcast_to(b.astype(jnp.float32), (16,))
    probs = _sc_pool(t, x3, lens_i, b16)
    return probs.reshape(_B, 1)


# speedup vs baseline: 10.7594x; 1.1293x over previous
import jax
import jax.numpy as jnp
from jax import lax
from jax.experimental import pallas as pl
from jax.experimental.pallas import tpu as pltpu
from jax.experimental.pallas import tpu_sc as plsc

_B = 4096
_L = 50
_V = 100000
_D = 64
_NC = 2
_NS = 16
_NW = _NC * _NS
_RPW = _B // _NW
_NG = _RPW // 16
_VB = 4096
_NVB = 25
_TR = _NVB * _VB // 128


def _tc_matvec_body(x_ref, w_ref, o_ref):
    x = x_ref[...]
    w = w_ref[...]
    o = lax.dot_general(w, x, (((1,), (1,)), ((), ())),
                        preferred_element_type=jnp.float32)
    o_ref[...] = o.reshape(_VB // 128, 128)


def _tc_matvec(emb_table, W):
    return pl.pallas_call(
        _tc_matvec_body,
        grid=(_NVB,),
        in_specs=[
            pl.BlockSpec((_VB, _D), lambda i: (i, 0)),
            pl.BlockSpec((1, _D), lambda i: (0, 0)),
        ],
        out_specs=pl.BlockSpec((_VB // 128, 128), lambda i: (i, 0)),
        out_shape=jax.ShapeDtypeStruct((_TR, 128), jnp.float32),
    )(emb_table, W)


def _sc_pool_body(t_hbm, x_hbm, lens_hbm, b_hbm, out_hbm,
                  t_v, x_v, lens_v, b_v, out_v):
    c = lax.axis_index("c")
    s = lax.axis_index("s")
    wid = s * _NC + c
    base = wid * _RPW
    pltpu.sync_copy(t_hbm, t_v)
    pltpu.sync_copy(x_hbm.at[pl.ds(base * _L, _RPW * _L)], x_v)
    pltpu.sync_copy(lens_hbm.at[pl.ds(base, _RPW)], lens_v)
    pltpu.sync_copy(b_hbm, b_v)
    bvec = b_v[...]
    riota = lax.iota(jnp.int32, 16) * _L
    lens_g = [lens_v[pl.ds(g * 16, 16)] for g in range(_NG)]
    base_g = [riota + g * 16 * _L for g in range(_NG)]

    def body(j, accs):
        new = []
        for g in range(_NG):
            xi = plsc.load_gather(x_v, [base_g[g] + j])
            vals = plsc.load_gather(t_v, [xi >> 7, xi & 127])
            mask = j < lens_g[g]
            new.append(accs[g] + jnp.where(mask, vals, 0.0))
        return tuple(new)

    accs = lax.fori_loop(
        0, _L, body,
        tuple(jnp.zeros((16,), jnp.float32) for _ in range(_NG)))
    for g in range(_NG):
        score = accs[g] / lens_g[g].astype(jnp.float32) + bvec
        out_v[pl.ds(g * 16, 16)] = 1.0 / (1.0 + jnp.exp(-score))
    pltpu.sync_copy(out_v, out_hbm.at[pl.ds(base, _RPW)])


_sc_pool = pl.kernel(
    _sc_pool_body,
    out_type=jax.ShapeDtypeStruct((_B,), jnp.float32),
    mesh=plsc.VectorSubcoreMesh(core_axis_name="c", subcore_axis_name="s",
                                num_cores=_NC, num_subcores=_NS),
    compiler_params=pltpu.CompilerParams(needs_layout_passes=False),
    scratch_types=[
        pltpu.VMEM((_TR, 128), jnp.float32),
        pltpu.VMEM((_RPW * _L,), jnp.int32),
        pltpu.VMEM((_RPW,), jnp.int32),
        pltpu.VMEM((16,), jnp.float32),
        pltpu.VMEM((_RPW,), jnp.float32),
    ],
)


def kernel(X, lens, emb_table, W, b):
    t2 = _tc_matvec(emb_table, W)
    xf = X.astype(jnp.int32).reshape(_B * _L)
    lens_i = lens.astype(jnp.int32)
    b16 = jnp.broadcast_to(b.astype(jnp.float32), (16,))
    probs = _sc_pool(t2, xf, lens_i, b16)
    return probs.reshape(_B, 1)


# speedup vs baseline: 20.5481x; 1.9098x over previous
import jax
import jax.numpy as jnp
from jax import lax
from jax.experimental import pallas as pl
from jax.experimental.pallas import tpu as pltpu
from jax.experimental.pallas import tpu_sc as plsc

_B = 4096
_L = 50
_V = 100000
_D = 64
_NC = 2
_NS = 16
_NW = _NC * _NS
_RPW = _B // _NW
_NG = _RPW // 16
_VB = 4096
_NVB = 25
_TR = _NVB * _VB // 128


def _tc_matvec_body(xt_ref, w_ref, o_ref):
    xt = xt_ref[...]
    w = w_ref[...]
    o = lax.dot_general(w, xt, (((1,), (0,)), ((), ())),
                        preferred_element_type=jnp.float32)
    o_ref[...] = o.reshape(_VB // 128, 128)


def _tc_matvec(emb_t, W):
    return pl.pallas_call(
        _tc_matvec_body,
        grid=(_NVB,),
        in_specs=[
            pl.BlockSpec((_D, _VB), lambda i: (0, i)),
            pl.BlockSpec((1, _D), lambda i: (0, 0)),
        ],
        out_specs=pl.BlockSpec((_VB // 128, 128), lambda i: (i, 0)),
        out_shape=jax.ShapeDtypeStruct((_TR, 128), jnp.float32),
    )(emb_t, W)


def _sc_pool_body(t_hbm, x_hbm, lens_hbm, b_hbm, out_hbm,
                  t_v, x_v, lens_v, b_v, out_v):
    c = lax.axis_index("c")
    s = lax.axis_index("s")
    wid = s * _NC + c
    base = wid * _RPW
    pltpu.sync_copy(t_hbm, t_v)
    pltpu.sync_copy(x_hbm.at[:, pl.ds(base, _RPW)], x_v)
    pltpu.sync_copy(lens_hbm.at[pl.ds(base, _RPW)], lens_v)
    pltpu.sync_copy(b_hbm, b_v)
    bvec = b_v[...]
    lens_g = [lens_v[pl.ds(g * 16, 16)] for g in range(_NG)]

    def body(j, accs):
        new = []
        for g in range(_NG):
            xi = x_v[j, pl.ds(g * 16, 16)]
            vals = plsc.load_gather(t_v, [xi >> 7, xi & 127])
            mask = j < lens_g[g]
            new.append(accs[g] + jnp.where(mask, vals, 0.0))
        return tuple(new)

    accs = lax.fori_loop(
        0, _L, body,
        tuple(jnp.zeros((16,), jnp.float32) for _ in range(_NG)))
    for g in range(_NG):
        score = accs[g] / lens_g[g].astype(jnp.float32) + bvec
        out_v[pl.ds(g * 16, 16)] = 1.0 / (1.0 + jnp.exp(-score))
    pltpu.sync_copy(out_v, out_hbm.at[pl.ds(base, _RPW)])


_sc_pool = pl.kernel(
    _sc_pool_body,
    out_type=jax.ShapeDtypeStruct((_B,), jnp.float32),
    mesh=plsc.VectorSubcoreMesh(core_axis_name="c", subcore_axis_name="s",
                                num_cores=_NC, num_subcores=_NS),
    compiler_params=pltpu.CompilerParams(needs_layout_passes=False),
    scratch_types=[
        pltpu.VMEM((_TR, 128), jnp.float32),
        pltpu.VMEM((_L, _RPW), jnp.int32),
        pltpu.VMEM((_RPW,), jnp.int32),
        pltpu.VMEM((16,), jnp.float32),
        pltpu.VMEM((_RPW,), jnp.float32),
    ],
)


def kernel(X, lens, emb_table, W, b):
    t2 = _tc_matvec(emb_table.T, W)
    xt = X.astype(jnp.int32).T
    lens_i = lens.astype(jnp.int32)
    b16 = jnp.broadcast_to(b.astype(jnp.float32), (16,))
    probs = _sc_pool(t2, xt, lens_i, b16)
    return probs.reshape(_B, 1)


# speedup vs baseline: 25.7815x; 1.2547x over previous
import jax
import jax.numpy as jnp
from jax import lax
from jax.experimental import pallas as pl
from jax.experimental.pallas import tpu as pltpu
from jax.experimental.pallas import tpu_sc as plsc

_B = 4096
_L = 50
_V = 100000
_D = 64
_NC = 2
_NS = 16
_NW = _NC * _NS
_RPW = _B // _NW
_NG = _RPW // 16
_VB = 16384
_NVB = 7
_TR = _NVB * _VB // 128


def _tc_matvec_body(xt_ref, w_ref, o_ref):
    xt = xt_ref[...]
    w = w_ref[...]
    o = lax.dot_general(w, xt, (((1,), (0,)), ((), ())),
                        preferred_element_type=jnp.float32)
    o_ref[...] = o.reshape(_VB // 128, 128)


def _tc_matvec(emb_t, W):
    return pl.pallas_call(
        _tc_matvec_body,
        grid=(_NVB,),
        in_specs=[
            pl.BlockSpec((_D, _VB), lambda i: (0, i)),
            pl.BlockSpec((1, _D), lambda i: (0, 0)),
        ],
        out_specs=pl.BlockSpec((_VB // 128, 128), lambda i: (i, 0)),
        out_shape=jax.ShapeDtypeStruct((_TR, 128), jnp.float32),
    )(emb_t, W)


def _sc_pool_body(t_hbm, x_hbm, lens_hbm, b_hbm, out_hbm,
                  t_v, x_v, lens_v, b_v, out_v, sem_t, sem_x, sem_l, sem_b):
    c = lax.axis_index("c")
    s = lax.axis_index("s")
    wid = s * _NC + c
    base = wid * _RPW
    ct = pltpu.async_copy(t_hbm, t_v, sem_t)
    cx = pltpu.async_copy(x_hbm.at[:, pl.ds(base, _RPW)], x_v, sem_x)
    cl = pltpu.async_copy(lens_hbm.at[pl.ds(base, _RPW)], lens_v, sem_l)
    cb = pltpu.async_copy(b_hbm, b_v, sem_b)
    cx.wait()
    cl.wait()
    cb.wait()
    ct.wait()
    bvec = b_v[...]
    lens_g = [lens_v[pl.ds(g * 16, 16)] for g in range(_NG)]

    def body(j, accs):
        new = []
        for g in range(_NG):
            xi = x_v[j, pl.ds(g * 16, 16)]
            vals = plsc.load_gather(t_v, [xi >> 7, xi & 127])
            mask = j < lens_g[g]
            new.append(accs[g] + jnp.where(mask, vals, 0.0))
        return tuple(new)

    accs = lax.fori_loop(
        0, _L, body,
        tuple(jnp.zeros((16,), jnp.float32) for _ in range(_NG)))
    for g in range(_NG):
        score = accs[g] / lens_g[g].astype(jnp.float32) + bvec
        out_v[pl.ds(g * 16, 16)] = 1.0 / (1.0 + jnp.exp(-score))
    pltpu.sync_copy(out_v, out_hbm.at[pl.ds(base, _RPW)])


_sc_pool = pl.kernel(
    _sc_pool_body,
    out_type=jax.ShapeDtypeStruct((_B,), jnp.float32),
    mesh=plsc.VectorSubcoreMesh(core_axis_name="c", subcore_axis_name="s",
                                num_cores=_NC, num_subcores=_NS),
    compiler_params=pltpu.CompilerParams(needs_layout_passes=False),
    scratch_types=[
        pltpu.VMEM((_TR, 128), jnp.float32),
        pltpu.VMEM((_L, _RPW), jnp.int32),
        pltpu.VMEM((_RPW,), jnp.int32),
        pltpu.VMEM((16,), jnp.float32),
        pltpu.VMEM((_RPW,), jnp.float32),
        pltpu.SemaphoreType.DMA,
        pltpu.SemaphoreType.DMA,
        pltpu.SemaphoreType.DMA,
        pltpu.SemaphoreType.DMA,
    ],
)


def kernel(X, lens, emb_table, W, b):
    t2 = _tc_matvec(emb_table.T, W)
    xt = X.astype(jnp.int32).T
    lens_i = lens.astype(jnp.int32)
    b16 = jnp.broadcast_to(b.astype(jnp.float32), (16,))
    probs = _sc_pool(t2, xt, lens_i, b16)
    return probs.reshape(_B, 1)


# speedup vs baseline: 25.8207x; 1.0015x over previous
import jax
import jax.numpy as jnp
from jax import lax
from jax.experimental import pallas as pl
from jax.experimental.pallas import tpu as pltpu
from jax.experimental.pallas import tpu_sc as plsc

_B = 4096
_L = 50
_V = 100000
_D = 64
_NC = 2
_NS = 16
_NW = _NC * _NS
_RPW = _B // _NW
_NG = _RPW // 16
_VB = 16384
_NVB = 7
_TR = _NVB * _VB // 128


def _tc_matvec_body(xt_ref, w_ref, o_ref):
    xt = xt_ref[...]
    w = w_ref[...]
    o = lax.dot_general(w, xt, (((1,), (0,)), ((), ())),
                        preferred_element_type=jnp.float32)
    o_ref[...] = o.reshape(_VB // 128, 128)


def _tc_matvec(emb_t, W):
    return pl.pallas_call(
        _tc_matvec_body,
        grid=(_NVB,),
        in_specs=[
            pl.BlockSpec((_D, _VB), lambda i: (0, i)),
            pl.BlockSpec((1, _D), lambda i: (0, 0)),
        ],
        out_specs=pl.BlockSpec((_VB // 128, 128), lambda i: (i, 0)),
        out_shape=jax.ShapeDtypeStruct((_TR, 128), jnp.float32),
    )(emb_t, W)


def _sc_pool_body(t_hbm, x_hbm, lens_hbm, b_hbm, out_hbm,
                  t_v, x_v, lens_v, b_v, out_v, sem_t, sem_x, sem_l, sem_b):
    c = lax.axis_index("c")
    s = lax.axis_index("s")
    wid = s * _NC + c
    base = wid * _RPW
    ct = pltpu.async_copy(t_hbm, t_v, sem_t)
    cx = pltpu.async_copy(x_hbm.at[:, pl.ds(base, _RPW)], x_v, sem_x)
    cl = pltpu.async_copy(lens_hbm.at[pl.ds(base, _RPW)], lens_v, sem_l)
    cb = pltpu.async_copy(b_hbm, b_v, sem_b)
    cx.wait()
    cl.wait()
    cb.wait()
    ct.wait()
    bvec = b_v[...]
    lens_g = [lens_v[pl.ds(g * 16, 16)] for g in range(_NG)]

    def step(j, accs):
        new = []
        for g in range(_NG):
            xi = x_v[j, pl.ds(g * 16, 16)]
            vals = plsc.load_gather(t_v, [xi])
            mask = j < lens_g[g]
            new.append(accs[g] + jnp.where(mask, vals, 0.0))
        return tuple(new)

    def body(i, accs):
        return step(i * 2 + 1, step(i * 2, accs))

    accs = lax.fori_loop(
        0, _L // 2, body,
        tuple(jnp.zeros((16,), jnp.float32) for _ in range(_NG)))
    for g in range(_NG):
        score = accs[g] / lens_g[g].astype(jnp.float32) + bvec
        out_v[pl.ds(g * 16, 16)] = 1.0 / (1.0 + jnp.exp(-score))
    pltpu.sync_copy(out_v, out_hbm.at[pl.ds(base, _RPW)])


_sc_pool = pl.kernel(
    _sc_pool_body,
    out_type=jax.ShapeDtypeStruct((_B,), jnp.float32),
    mesh=plsc.VectorSubcoreMesh(core_axis_name="c", subcore_axis_name="s",
                                num_cores=_NC, num_subcores=_NS),
    compiler_params=pltpu.CompilerParams(needs_layout_passes=False),
    scratch_types=[
        pltpu.VMEM((_TR * 128,), jnp.float32),
        pltpu.VMEM((_L, _RPW), jnp.int32),
        pltpu.VMEM((_RPW,), jnp.int32),
        pltpu.VMEM((16,), jnp.float32),
        pltpu.VMEM((_RPW,), jnp.float32),
        pltpu.SemaphoreType.DMA,
        pltpu.SemaphoreType.DMA,
        pltpu.SemaphoreType.DMA,
        pltpu.SemaphoreType.DMA,
    ],
)


def kernel(X, lens, emb_table, W, b):
    t2 = _tc_matvec(emb_table.T, W).reshape(_TR * 128)
    xt = X.astype(jnp.int32).T
    lens_i = lens.astype(jnp.int32)
    b16 = jnp.broadcast_to(b.astype(jnp.float32), (16,))
    probs = _sc_pool(t2, xt, lens_i, b16)
    return probs.reshape(_B, 1)


# speedup vs baseline: 25.8393x; 1.0007x over previous
import jax
import jax.numpy as jnp
from jax import lax
from jax.experimental import pallas as pl
from jax.experimental.pallas import tpu as pltpu
from jax.experimental.pallas import tpu_sc as plsc

_B = 4096
_L = 50
_V = 100000
_D = 64
_NC = 2
_NS = 16
_NW = _NC * _NS
_RPW = _B // _NW
_NG = _RPW // 16
_VB = 16384
_NVB = 7
_TR = _NVB * _VB // 128


def _tc_matvec_body(xt_ref, w_ref, o_ref):
    xt = xt_ref[...]
    w = w_ref[...]
    o = lax.dot_general(w, xt, (((1,), (0,)), ((), ())),
                        preferred_element_type=jnp.float32)
    o_ref[...] = o.reshape(_VB // 128, 128)


def _tc_matvec(emb_t, W):
    return pl.pallas_call(
        _tc_matvec_body,
        grid=(_NVB,),
        in_specs=[
            pl.BlockSpec((_D, _VB), lambda i: (0, i)),
            pl.BlockSpec((1, _D), lambda i: (0, 0)),
        ],
        out_specs=pl.BlockSpec((_VB // 128, 128), lambda i: (i, 0)),
        out_shape=jax.ShapeDtypeStruct((_TR, 128), jnp.float32),
    )(emb_t, W)


def _sc_pool_body(t_hbm, x_hbm, lens_hbm, b_hbm, out_hbm,
                  t_v, x_v, lens_v, b_v, out_v, sem_t, sem_x, sem_l, sem_b):
    c = lax.axis_index("c")
    s = lax.axis_index("s")
    wid = s * _NC + c
    base = wid * _RPW
    nch = 4
    tch = _TR * 128 // nch
    cts = [pltpu.async_copy(t_hbm.at[pl.ds(k * tch, tch)],
                            t_v.at[pl.ds(k * tch, tch)], sem_t)
           for k in range(nch)]
    cx = pltpu.async_copy(x_hbm.at[:, pl.ds(base, _RPW)], x_v, sem_x)
    cl = pltpu.async_copy(lens_hbm.at[pl.ds(base, _RPW)], lens_v, sem_l)
    cb = pltpu.async_copy(b_hbm, b_v, sem_b)
    cx.wait()
    cl.wait()
    cb.wait()
    for ctc in cts:
        ctc.wait()
    bvec = b_v[...]
    lens_g = [lens_v[pl.ds(g * 16, 16)] for g in range(_NG)]

    def step(j, accs):
        new = []
        for g in range(_NG):
            xi = x_v[j, pl.ds(g * 16, 16)]
            vals = plsc.load_gather(t_v, [xi])
            mask = j < lens_g[g]
            new.append(accs[g] + jnp.where(mask, vals, 0.0))
        return tuple(new)

    def body(i, accs):
        return step(i * 2 + 1, step(i * 2, accs))

    accs = lax.fori_loop(
        0, _L // 2, body,
        tuple(jnp.zeros((16,), jnp.float32) for _ in range(_NG)))
    for g in range(_NG):
        score = accs[g] / lens_g[g].astype(jnp.float32) + bvec
        out_v[pl.ds(g * 16, 16)] = 1.0 / (1.0 + jnp.exp(-score))
    pltpu.sync_copy(out_v, out_hbm.at[pl.ds(base, _RPW)])


_sc_pool = pl.kernel(
    _sc_pool_body,
    out_type=jax.ShapeDtypeStruct((_B,), jnp.float32),
    mesh=plsc.VectorSubcoreMesh(core_axis_name="c", subcore_axis_name="s",
                                num_cores=_NC, num_subcores=_NS),
    compiler_params=pltpu.CompilerParams(needs_layout_passes=False),
    scratch_types=[
        pltpu.VMEM((_TR * 128,), jnp.float32),
        pltpu.VMEM((_L, _RPW), jnp.int32),
        pltpu.VMEM((_RPW,), jnp.int32),
        pltpu.VMEM((16,), jnp.float32),
        pltpu.VMEM((_RPW,), jnp.float32),
        pltpu.SemaphoreType.DMA,
        pltpu.SemaphoreType.DMA,
        pltpu.SemaphoreType.DMA,
        pltpu.SemaphoreType.DMA,
    ],
)


def kernel(X, lens, emb_table, W, b):
    t2 = _tc_matvec(emb_table.T, W).reshape(_TR * 128)
    xt = X.astype(jnp.int32).T
    lens_i = lens.astype(jnp.int32)
    b16 = jnp.broadcast_to(b.astype(jnp.float32), (16,))
    probs = _sc_pool(t2, xt, lens_i, b16)
    return probs.reshape(_B, 1)
